# baseline (speedup 1.0000x reference)
import functools

import jax
import jax.numpy as jnp
from jax import lax
from jax.experimental import pallas as pl
from jax.experimental.pallas import tpu as pltpu
from jax.experimental.pallas import tpu_sc as plsc

N = 10000
E = 320000
D = 128
NCORES = 2
NSUB = 16
NTILES = NCORES * NSUB
CHUNK = 128
NCHUNK = 80
EDGES_PER_TILE = CHUNK * NCHUNK
E_PAD = NTILES * EDGES_PER_TILE
NSH = 10240
BR = 1000
NB = 2
NROUND = NCHUNK // NB

_ZC5 = ((0, 128), (1, 128), (2, 128), (3, 128), (4, 128))


def _mesh():
    return plsc.VectorSubcoreMesh(core_axis_name="c", subcore_axis_name="s")


@functools.partial(
    pl.kernel,
    out_type=jax.ShapeDtypeStruct((NCORES, NSH), jnp.float32),
    scratch_types=[
        pltpu.VMEM((NCHUNK, CHUNK), jnp.int32),
        pltpu.VMEM((CHUNK,), jnp.float32),
        pltpu.VMEM((640,), jnp.float32),
        pltpu.VMEM_SHARED((NSH,), jnp.float32),
        pltpu.SemaphoreType.DMA,
    ],
    mesh=_mesh(),
)
def _deg_call(rp_hbm, deg_hbm, ridx_v, ones_v, zbuf, deg_sh, sem):
    c = lax.axis_index("c")
    s = lax.axis_index("s")
    wid = c * NSUB + s
    for j in range(CHUNK // 16):
        ones_v[pl.ds(j * 16, 16)] = jnp.ones((16,), jnp.float32)
    for j in range(640 // 16):
        zbuf[pl.ds(j * 16, 16)] = jnp.zeros((16,), jnp.float32)
    pltpu.sync_copy(zbuf, deg_sh.at[pl.ds(s * 640, 640)])
    pltpu.sync_copy(rp_hbm.at[pl.ds(wid * NCHUNK, NCHUNK)], ridx_v)
    plsc.subcore_barrier()

    def body(k, carry):
        pltpu.sync_copy(ones_v, deg_sh.at[ridx_v.at[k]], add=True)
        return carry

    lax.fori_loop(0, NCHUNK, body, 0)
    plsc.subcore_barrier()
    pltpu.sync_copy(deg_sh.at[pl.ds(s * 640, 640)],
                    deg_hbm.at[c, pl.ds(s * 640, 640)])


@functools.partial(
    pl.kernel,
    out_type=jax.ShapeDtypeStruct((NCORES, NSH, D), jnp.float32),
    scratch_types=[
        pltpu.VMEM((CHUNK,), jnp.int32),
        pltpu.VMEM((CHUNK,), jnp.int32),
        pltpu.VMEM((CHUNK,), jnp.int32),
        pltpu.VMEM((CHUNK,), jnp.int32),
        pltpu.VMEM((CHUNK,), jnp.int32),
        pltpu.VMEM((CHUNK,), jnp.int32),
        pltpu.VMEM((CHUNK,), jnp.int32),
        pltpu.VMEM((CHUNK,), jnp.int32),
        pltpu.VMEM((CHUNK, D), jnp.float32),
        pltpu.VMEM((CHUNK, D), jnp.float32),
        pltpu.VMEM_SHARED((NSH, D), jnp.float32),
        pltpu.SemaphoreType.DMA,
        pltpu.SemaphoreType.DMA,
        pltpu.SemaphoreType.DMA,
        pltpu.SemaphoreType.DMA,
        pltpu.SemaphoreType.DMA,
        pltpu.SemaphoreType.DMA,
        pltpu.SemaphoreType.DMA,
        pltpu.SemaphoreType.DMA,
        pltpu.SemaphoreType.DMA,
        pltpu.SemaphoreType.DMA,
    ],
    mesh=_mesh(),
)
def _msg_call(cp_hbm, rp_hbm, h_hbm, z_hbm, out_hbm,
              cb0, cb1, cb2, cb3, rb0, rb1, rb2, rb3, m0, m1, acc_sh,
              g0, g1, c0, c1, c2, c3, r0, r1, r2, r3):
    cbufs = (cb0, cb1, cb2, cb3)
    rbufs = (rb0, rb1, rb2, rb3)
    msgs = (m0, m1)
    gsems = (g0, g1)
    csems = (c0, c1, c2, c3)
    rsems = (r0, r1, r2, r3)
    c = lax.axis_index("c")
    s = lax.axis_index("s")
    wid = c * NSUB + s
    for k, sz in _ZC5:
        pltpu.sync_copy(z_hbm.at[pl.ds(0, sz)],
                        acc_sh.at[pl.ds(s * 640 + k * 128, sz)])
    base = wid * NCHUNK
    plsc.subcore_barrier()

    @pl.when(c == 1)
    def _():
        for i in range(4):
            pltpu.async_copy(cp_hbm.at[base + i], cbufs[i], csems[i])
            pltpu.async_copy(rp_hbm.at[base + i], rbufs[i], rsems[i])
        for j in range(2):
            pltpu.make_async_copy(cp_hbm.at[0], cbufs[j], csems[j]).wait()
            pltpu.async_copy(h_hbm.at[cbufs[j]], msgs[j], gsems[j])

    def body(k, carry):
        for j in range(4):
            kk = k * 4 + j
            m = j % 2
            j2 = (j + 2) % 4
            pltpu.make_async_copy(h_hbm.at[cbufs[j]], msgs[m], gsems[m]).wait()
            pltpu.make_async_copy(rp_hbm.at[0], rbufs[j], rsems[j]).wait()
            pltpu.sync_copy(msgs[m], acc_sh.at[rbufs[j]], add=True)

            @pl.when(kk + 4 < NCHUNK)
            def _():
                pltpu.async_copy(cp_hbm.at[base + kk + 4], cbufs[j], csems[j])
                pltpu.async_copy(rp_hbm.at[base + kk + 4], rbufs[j], rsems[j])

            @pl.when(kk + 2 < NCHUNK)
            def _():
                pltpu.make_async_copy(cp_hbm.at[0], cbufs[j2],
                                      csems[j2]).wait()
                pltpu.async_copy(h_hbm.at[cbufs[j2]], msgs[m], gsems[m])

        return carry

    @pl.when(c == 1)
    def _():
        lax.fori_loop(0, NCHUNK // 4, body, 0)
    plsc.subcore_barrier()
    for k, sz in _ZC5:
        r0 = s * 640 + k * 128
        pltpu.sync_copy(acc_sh.at[pl.ds(r0, sz)], out_hbm.at[c, pl.ds(r0, sz)])


def _mm_body(x_ref, w_ref, d0_ref, d1_ref, h_ref):
    deg = d0_ref[...] + d1_ref[...]
    dinv = jnp.where(deg > 0, lax.rsqrt(jnp.maximum(deg, 1e-12)), 0.0)
    h_ref[...] = jnp.dot(x_ref[...], w_ref[...],
                         preferred_element_type=jnp.float32) * dinv


def _fin_body(p0_ref, p1_ref, d0_ref, d1_ref, b_ref, o_ref):
    deg = d0_ref[...] + d1_ref[...]
    dinv = jnp.where(deg > 0, lax.rsqrt(jnp.maximum(deg, 1e-12)), 0.0)
    o_ref[...] = jnp.maximum((p0_ref[...] + p1_ref[...]) * dinv + b_ref[...],
                             0.0)


def kernel(x, edge_index, W, b):
    row = edge_index[0]
    col = edge_index[1]
    pad = E_PAD - E
    dummy = N + (jnp.arange(pad, dtype=jnp.int32) % (NSH - N))
    rp = jnp.concatenate([row, dummy])
    cp = jnp.concatenate([col, jnp.zeros((pad,), jnp.int32)])
    rp2 = rp.reshape(NTILES * NCHUNK, CHUNK)
    cp2 = cp.reshape(NTILES * NCHUNK, CHUNK)
    z128 = jnp.zeros((CHUNK, D), jnp.float32)

    deg2 = _deg_call(rp2)
    d0 = deg2[0, :N].reshape(N, 1)
    d1 = deg2[1, :N].reshape(N, 1)

    h = pl.pallas_call(
        _mm_body,
        grid=(N // BR,),
        in_specs=[
            pl.BlockSpec((BR, D), lambda i: (i, 0)),
            pl.BlockSpec((D, D), lambda i: (0, 0)),
            pl.BlockSpec((BR, 1), lambda i: (i, 0)),
            pl.BlockSpec((BR, 1), lambda i: (i, 0)),
        ],
        out_specs=pl.BlockSpec((BR, D), lambda i: (i, 0)),
        out_shape=jax.ShapeDtypeStruct((N, D), jnp.float32),
    )(x, W, d0, d1)

    parts = _msg_call(cp2, rp2, h, z128)

    out = pl.pallas_call(
        _fin_body,
        grid=(N // BR,),
        in_specs=[
            pl.BlockSpec((BR, D), lambda i: (i, 0)),
            pl.BlockSpec((BR, D), lambda i: (i, 0)),
            pl.BlockSpec((BR, 1), lambda i: (i, 0)),
            pl.BlockSpec((BR, 1), lambda i: (i, 0)),
            pl.BlockSpec((1, D), lambda i: (0, 0)),
        ],
        out_specs=pl.BlockSpec((BR, D), lambda i: (i, 0)),
        out_shape=jax.ShapeDtypeStruct((N, D), jnp.float32),
    )(parts[0, :N], parts[1, :N], d0, d1, b.reshape(1, D))
    return out


# speedup vs baseline: 2.7118x; 2.7118x over previous
import functools

import jax
import jax.numpy as jnp
from jax import lax
from jax.experimental import pallas as pl
from jax.experimental.pallas import tpu as pltpu
from jax.experimental.pallas import tpu_sc as plsc

N = 10000
E = 320000
D = 128
NCORES = 2
NSUB = 16
NTILES = NCORES * NSUB
CHUNK = 128
NCHUNK = 80
EDGES_PER_TILE = CHUNK * NCHUNK
E_PAD = NTILES * EDGES_PER_TILE
NSH = 10240
BR = 1000
NB = 2
NROUND = NCHUNK // NB

_ZC5 = ((0, 128), (1, 128), (2, 128), (3, 128), (4, 128))


def _mesh():
    return plsc.VectorSubcoreMesh(core_axis_name="c", subcore_axis_name="s")


@functools.partial(
    pl.kernel,
    out_type=jax.ShapeDtypeStruct((NCORES, NSH), jnp.float32),
    scratch_types=[
        pltpu.VMEM((NCHUNK, CHUNK), jnp.int32),
        pltpu.VMEM((CHUNK,), jnp.float32),
        pltpu.VMEM((640,), jnp.float32),
        pltpu.VMEM_SHARED((NSH,), jnp.float32),
        pltpu.SemaphoreType.DMA,
    ],
    mesh=_mesh(),
)
def _deg_call(rp_hbm, deg_hbm, ridx_v, ones_v, zbuf, deg_sh, sem):
    c = lax.axis_index("c")
    s = lax.axis_index("s")
    wid = c * NSUB + s
    for j in range(CHUNK // 16):
        ones_v[pl.ds(j * 16, 16)] = jnp.ones((16,), jnp.float32)
    for j in range(640 // 16):
        zbuf[pl.ds(j * 16, 16)] = jnp.zeros((16,), jnp.float32)
    pltpu.sync_copy(zbuf, deg_sh.at[pl.ds(s * 640, 640)])
    pltpu.sync_copy(rp_hbm.at[pl.ds(wid * NCHUNK, NCHUNK)], ridx_v)
    plsc.subcore_barrier()

    def body(k, carry):
        pltpu.sync_copy(ones_v, deg_sh.at[ridx_v.at[k]], add=True)
        return carry

    lax.fori_loop(0, NCHUNK, body, 0)
    plsc.subcore_barrier()
    pltpu.sync_copy(deg_sh.at[pl.ds(s * 640, 640)],
                    deg_hbm.at[c, pl.ds(s * 640, 640)])


@functools.partial(
    pl.kernel,
    out_type=jax.ShapeDtypeStruct((NCORES, NSH, D), jnp.float32),
    scratch_types=[
        pltpu.VMEM((CHUNK,), jnp.int32),
        pltpu.VMEM((CHUNK,), jnp.int32),
        pltpu.VMEM((CHUNK,), jnp.int32),
        pltpu.VMEM((CHUNK,), jnp.int32),
        pltpu.VMEM((CHUNK,), jnp.int32),
        pltpu.VMEM((CHUNK,), jnp.int32),
        pltpu.VMEM((CHUNK,), jnp.int32),
        pltpu.VMEM((CHUNK,), jnp.int32),
        pltpu.VMEM((CHUNK, D), jnp.float32),
        pltpu.VMEM((CHUNK, D), jnp.float32),
        pltpu.VMEM_SHARED((NSH, D), jnp.float32),
        pltpu.SemaphoreType.DMA,
        pltpu.SemaphoreType.DMA,
        pltpu.SemaphoreType.DMA,
        pltpu.SemaphoreType.DMA,
        pltpu.SemaphoreType.DMA,
        pltpu.SemaphoreType.DMA,
        pltpu.SemaphoreType.DMA,
        pltpu.SemaphoreType.DMA,
        pltpu.SemaphoreType.DMA,
        pltpu.SemaphoreType.DMA,
    ],
    mesh=_mesh(),
)
def _msg_call(cp_hbm, rp_hbm, h_hbm, z_hbm, out_hbm,
              cb0, cb1, cb2, cb3, rb0, rb1, rb2, rb3, m0, m1, acc_sh,
              g0, g1, c0, c1, c2, c3, r0, r1, r2, r3):
    cbufs = (cb0, cb1, cb2, cb3)
    rbufs = (rb0, rb1, rb2, rb3)
    msgs = (m0, m1)
    gsems = (g0, g1)
    csems = (c0, c1, c2, c3)
    rsems = (r0, r1, r2, r3)
    c = lax.axis_index("c")
    s = lax.axis_index("s")
    wid = c * NSUB + s
    for k, sz in _ZC5:
        pltpu.sync_copy(z_hbm.at[pl.ds(0, sz)],
                        acc_sh.at[pl.ds(s * 640 + k * 128, sz)])
    base = wid * NCHUNK
    for i in range(4):
        pltpu.async_copy(cp_hbm.at[base + i], cbufs[i], csems[i])
        pltpu.async_copy(rp_hbm.at[base + i], rbufs[i], rsems[i])
    plsc.subcore_barrier()
    for j in range(2):
        pltpu.make_async_copy(cp_hbm.at[0], cbufs[j], csems[j]).wait()
        pltpu.async_copy(h_hbm.at[cbufs[j]], msgs[j], gsems[j])

    def body(k, carry):
        for j in range(4):
            kk = k * 4 + j
            m = j % 2
            j2 = (j + 2) % 4
            pltpu.make_async_copy(h_hbm.at[cbufs[j]], msgs[m], gsems[m]).wait()
            pltpu.make_async_copy(rp_hbm.at[0], rbufs[j], rsems[j]).wait()
            pltpu.sync_copy(msgs[m], acc_sh.at[rbufs[j]], add=True)

            @pl.when(kk + 4 < NCHUNK)
            def _():
                pltpu.async_copy(cp_hbm.at[base + kk + 4], cbufs[j], csems[j])
                pltpu.async_copy(rp_hbm.at[base + kk + 4], rbufs[j], rsems[j])

            @pl.when(kk + 2 < NCHUNK)
            def _():
                pltpu.make_async_copy(cp_hbm.at[0], cbufs[j2],
                                      csems[j2]).wait()
                pltpu.async_copy(h_hbm.at[cbufs[j2]], msgs[m], gsems[m])

        return carry

    lax.fori_loop(0, NCHUNK // 4, body, 0)
    plsc.subcore_barrier()
    for k, sz in _ZC5:
        r0 = s * 640 + k * 128
        pltpu.sync_copy(acc_sh.at[pl.ds(r0, sz)], out_hbm.at[c, pl.ds(r0, sz)])


def _mm_body(x_ref, w_ref, d0_ref, d1_ref, h_ref):
    deg = d0_ref[...] + d1_ref[...]
    dinv = jnp.where(deg > 0, lax.rsqrt(jnp.maximum(deg, 1e-12)), 0.0)
    h_ref[...] = jnp.dot(x_ref[...], w_ref[...],
                         preferred_element_type=jnp.float32) * dinv


def _fin_body(p0_ref, p1_ref, d0_ref, d1_ref, b_ref, o_ref):
    deg = d0_ref[...] + d1_ref[...]
    dinv = jnp.where(deg > 0, lax.rsqrt(jnp.maximum(deg, 1e-12)), 0.0)
    o_ref[...] = jnp.maximum((p0_ref[...] + p1_ref[...]) * dinv + b_ref[...],
                             0.0)


def kernel(x, edge_index, W, b):
    row = edge_index[0]
    col = edge_index[1]
    pad = E_PAD - E
    dummy = N + (jnp.arange(pad, dtype=jnp.int32) % (NSH - N))
    rp = jnp.concatenate([row, dummy])
    dummy_c = jnp.arange(pad, dtype=jnp.int32) % N
    cp = jnp.concatenate([col, dummy_c])
    rp2 = rp.reshape(NTILES * NCHUNK, CHUNK)
    cp2 = cp.reshape(NTILES * NCHUNK, CHUNK)
    z128 = jnp.zeros((CHUNK, D), jnp.float32)

    deg2 = _deg_call(rp2)
    d0 = deg2[0, :N].reshape(N, 1)
    d1 = deg2[1, :N].reshape(N, 1)

    h = pl.pallas_call(
        _mm_body,
        grid=(N // BR,),
        in_specs=[
            pl.BlockSpec((BR, D), lambda i: (i, 0)),
            pl.BlockSpec((D, D), lambda i: (0, 0)),
            pl.BlockSpec((BR, 1), lambda i: (i, 0)),
            pl.BlockSpec((BR, 1), lambda i: (i, 0)),
        ],
        out_specs=pl.BlockSpec((BR, D), lambda i: (i, 0)),
        out_shape=jax.ShapeDtypeStruct((N, D), jnp.float32),
    )(x, W, d0, d1)

    parts = _msg_call(cp2, rp2, h, z128)

    out = pl.pallas_call(
        _fin_body,
        grid=(N // BR,),
        in_specs=[
            pl.BlockSpec((BR, D), lambda i: (i, 0)),
            pl.BlockSpec((BR, D), lambda i: (i, 0)),
            pl.BlockSpec((BR, 1), lambda i: (i, 0)),
            pl.BlockSpec((BR, 1), lambda i: (i, 0)),
            pl.BlockSpec((1, D), lambda i: (0, 0)),
        ],
        out_specs=pl.BlockSpec((BR, D), lambda i: (i, 0)),
        out_shape=jax.ShapeDtypeStruct((N, D), jnp.float32),
    )(parts[0, :N], parts[1, :N], d0, d1, b.reshape(1, D))
    return out


# speedup vs baseline: 2.8865x; 1.0644x over previous
import functools

import jax
import jax.numpy as jnp
from jax import lax
from jax.experimental import pallas as pl
from jax.experimental.pallas import tpu as pltpu
from jax.experimental.pallas import tpu_sc as plsc

N = 10000
E = 320000
D = 128
NCORES = 2
NSUB = 16
NTILES = NCORES * NSUB
CHUNK = 128
NCHUNK = 80
EDGES_PER_TILE = CHUNK * NCHUNK
E_PAD = NTILES * EDGES_PER_TILE
NSH = 10240
BR = 1000
NB = 2
NROUND = NCHUNK // NB

_ZC5 = ((0, 128), (1, 128), (2, 128), (3, 128), (4, 128))


def _mesh():
    return plsc.VectorSubcoreMesh(core_axis_name="c", subcore_axis_name="s")


@functools.partial(
    pl.kernel,
    out_type=jax.ShapeDtypeStruct((NCORES, NSH), jnp.float32),
    scratch_types=[
        pltpu.VMEM((NCHUNK, CHUNK), jnp.int32),
        pltpu.VMEM((CHUNK,), jnp.float32),
        pltpu.VMEM((640,), jnp.float32),
        pltpu.VMEM_SHARED((NSH,), jnp.float32),
        pltpu.SemaphoreType.DMA,
    ],
    mesh=_mesh(),
)
def _deg_call(rp_hbm, deg_hbm, ridx_v, ones_v, zbuf, deg_sh, sem):
    c = lax.axis_index("c")
    s = lax.axis_index("s")
    wid = c * NSUB + s
    for j in range(CHUNK // 16):
        ones_v[pl.ds(j * 16, 16)] = jnp.ones((16,), jnp.float32)
    for j in range(640 // 16):
        zbuf[pl.ds(j * 16, 16)] = jnp.zeros((16,), jnp.float32)
    pltpu.sync_copy(zbuf, deg_sh.at[pl.ds(s * 640, 640)])
    pltpu.sync_copy(rp_hbm.at[pl.ds(wid * NCHUNK, NCHUNK)], ridx_v)
    plsc.subcore_barrier()

    def body(k, carry):
        pltpu.sync_copy(ones_v, deg_sh.at[ridx_v.at[k]], add=True)
        return carry

    lax.fori_loop(0, NCHUNK, body, 0)
    plsc.subcore_barrier()
    pltpu.sync_copy(deg_sh.at[pl.ds(s * 640, 640)],
                    deg_hbm.at[c, pl.ds(s * 640, 640)])


@functools.partial(
    pl.kernel,
    out_type=jax.ShapeDtypeStruct((NCORES, NSH, D), jnp.float32),
    scratch_types=[
        pltpu.VMEM((CHUNK,), jnp.int32),
        pltpu.VMEM((CHUNK,), jnp.int32),
        pltpu.VMEM((CHUNK,), jnp.int32),
        pltpu.VMEM((CHUNK,), jnp.int32),
        pltpu.VMEM((CHUNK,), jnp.int32),
        pltpu.VMEM((CHUNK,), jnp.int32),
        pltpu.VMEM((CHUNK,), jnp.int32),
        pltpu.VMEM((CHUNK,), jnp.int32),
        pltpu.VMEM((CHUNK, D), jnp.float32),
        pltpu.VMEM((CHUNK, D), jnp.float32),
        pltpu.VMEM_SHARED((NSH, D), jnp.float32),
        pltpu.SemaphoreType.DMA,
        pltpu.SemaphoreType.DMA,
        pltpu.SemaphoreType.DMA,
        pltpu.SemaphoreType.DMA,
        pltpu.SemaphoreType.DMA,
        pltpu.SemaphoreType.DMA,
        pltpu.SemaphoreType.DMA,
        pltpu.SemaphoreType.DMA,
        pltpu.SemaphoreType.DMA,
        pltpu.SemaphoreType.DMA,
    ],
    mesh=_mesh(),
)
def _msg_call(cp_hbm, rp_hbm, h_hbm, z_hbm, out_hbm,
              cb0, cb1, cb2, cb3, rb0, rb1, rb2, rb3, m0, m1, acc_sh,
              g0, g1, c0, c1, c2, c3, r0, r1, r2, r3):
    cbufs = (cb0, cb1, cb2, cb3)
    rbufs = (rb0, rb1, rb2, rb3)
    msgs = (m0, m1)
    gsems = (g0, g1)
    csems = (c0, c1, c2, c3)
    rsems = (r0, r1, r2, r3)
    c = lax.axis_index("c")
    s = lax.axis_index("s")
    wid = c * NSUB + s
    for k, sz in _ZC5:
        pltpu.sync_copy(z_hbm.at[pl.ds(0, sz)],
                        acc_sh.at[pl.ds(s * 640 + k * 128, sz)])
    base = wid * NCHUNK
    for i in range(4):
        pltpu.async_copy(cp_hbm.at[base + i], cbufs[i], csems[i])
        pltpu.async_copy(rp_hbm.at[base + i], rbufs[i], rsems[i])
    plsc.subcore_barrier()
    for j in range(2):
        pltpu.make_async_copy(cp_hbm.at[0], cbufs[j], csems[j]).wait()
        pltpu.async_copy(h_hbm.at[cbufs[j]], msgs[j], gsems[j])

    def body(k, carry):
        for j in range(4):
            kk = k * 4 + j
            m = j % 2
            j2 = (j + 2) % 4
            pltpu.make_async_copy(h_hbm.at[cbufs[j]], msgs[m], gsems[m]).wait()
            pltpu.make_async_copy(rp_hbm.at[0], rbufs[j], rsems[j]).wait()
            pltpu.sync_copy(msgs[m], acc_sh.at[rbufs[j]], add=True)

            @pl.when(kk + 4 < NCHUNK)
            def _():
                pltpu.async_copy(cp_hbm.at[base + kk + 4], cbufs[j], csems[j])
                pltpu.async_copy(rp_hbm.at[base + kk + 4], rbufs[j], rsems[j])

            @pl.when(kk + 2 < NCHUNK)
            def _():
                pltpu.make_async_copy(cp_hbm.at[0], cbufs[j2],
                                      csems[j2]).wait()
                pltpu.async_copy(h_hbm.at[cbufs[j2]], msgs[m], gsems[m])

        return carry

    lax.fori_loop(0, NCHUNK // 4, body, 0)
    plsc.subcore_barrier()
    for k, sz in _ZC5:
        r0 = s * 640 + k * 128
        pltpu.sync_copy(acc_sh.at[pl.ds(r0, sz)], out_hbm.at[c, pl.ds(r0, sz)])


def _mm_body(x_ref, w_ref, d0_ref, d1_ref, h_ref):
    deg = d0_ref[...] + d1_ref[...]
    dinv = jnp.where(deg > 0, lax.rsqrt(jnp.maximum(deg, 1e-12)), 0.0)
    h_ref[...] = jnp.dot(x_ref[...], w_ref[...],
                         preferred_element_type=jnp.float32) * dinv


def _fin_body(p0_ref, p1_ref, d0_ref, d1_ref, b_ref, o_ref):
    deg = d0_ref[...] + d1_ref[...]
    dinv = jnp.where(deg > 0, lax.rsqrt(jnp.maximum(deg, 1e-12)), 0.0)
    o_ref[...] = jnp.maximum((p0_ref[0] + p1_ref[0]) * dinv + b_ref[...],
                             0.0)


def kernel(x, edge_index, W, b):
    row = edge_index[0]
    col = edge_index[1]
    pad = E_PAD - E
    dummy = N + (jnp.arange(pad, dtype=jnp.int32) % (NSH - N))
    rp = jnp.concatenate([row, dummy])
    dummy_c = jnp.arange(pad, dtype=jnp.int32) % N
    cp = jnp.concatenate([col, dummy_c])
    rp2 = rp.reshape(NTILES * NCHUNK, CHUNK)
    cp2 = cp.reshape(NTILES * NCHUNK, CHUNK)
    z128 = jnp.zeros((CHUNK, D), jnp.float32)

    deg2 = _deg_call(rp2)
    d0 = deg2[0, :N].reshape(N, 1)
    d1 = deg2[1, :N].reshape(N, 1)

    h = pl.pallas_call(
        _mm_body,
        grid=(N // BR,),
        in_specs=[
            pl.BlockSpec((BR, D), lambda i: (i, 0)),
            pl.BlockSpec((D, D), lambda i: (0, 0)),
            pl.BlockSpec((BR, 1), lambda i: (i, 0)),
            pl.BlockSpec((BR, 1), lambda i: (i, 0)),
        ],
        out_specs=pl.BlockSpec((BR, D), lambda i: (i, 0)),
        out_shape=jax.ShapeDtypeStruct((N, D), jnp.float32),
    )(x, W, d0, d1)

    parts = _msg_call(cp2, rp2, h, z128)

    out = pl.pallas_call(
        _fin_body,
        grid=(N // BR,),
        in_specs=[
            pl.BlockSpec((1, BR, D), lambda i: (0, i, 0)),
            pl.BlockSpec((1, BR, D), lambda i: (1, i, 0)),
            pl.BlockSpec((BR, 1), lambda i: (i, 0)),
            pl.BlockSpec((BR, 1), lambda i: (i, 0)),
            pl.BlockSpec((1, D), lambda i: (0, 0)),
        ],
        out_specs=pl.BlockSpec((BR, D), lambda i: (i, 0)),
        out_shape=jax.ShapeDtypeStruct((N, D), jnp.float32),
    )(parts, parts, d0, d1, b.reshape(1, D))
    return out
